# Initial kernel scaffold; baseline (speedup 1.0000x reference)
import jax
import jax.numpy as jnp
from jax.experimental import pallas as pl


def kernel(feat, edge_index_0, edge_index_1, edge_index_2, W0, al0, ar0, b0, R0, W1, al1, ar1, b1, R1):
    raise NotImplementedError("write your pallas kernel here")


# speedup vs baseline: 7.3189x; 7.3189x over previous
import functools

import jax
import jax.numpy as jnp
from jax import lax
from jax.experimental import pallas as pl
from jax.experimental.pallas import tpu as pltpu
from jax.experimental.pallas import tpu_sc as plsc

F32 = jnp.float32
PREC = lax.Precision.HIGHEST

_N = 50000
_E = 800000
_G = 3
_IN = 14
_H = 2
_D0 = 64
_D1 = 16
_HD0 = _H * _D0
_HD1 = _H * _D1

_BM = 1024
_NBLK = 49
_NP = _BM * _NBLK

_NC = 2
_NS = 16
_ROWS_T = _NP // _NS
_EDG_T = _E // _NS
_B = 80
_NBATCH = _EDG_T // _B


def _pre_body(feat, W, Al, Ar, R, ft_o, el_o, er_o, res_o):
    x = feat[...]
    ft = jnp.dot(x, W[...], preferred_element_type=F32, precision=PREC)
    ft_o[...] = ft
    el_o[...] = jnp.dot(ft, Al[...], preferred_element_type=F32, precision=PREC)
    er_o[...] = jnp.dot(ft, Ar[...], preferred_element_type=F32, precision=PREC)
    res_o[...] = jnp.dot(x, R[...], preferred_element_type=F32, precision=PREC)


def _tc_pre(featp, W, Al, Ar, R):
    return pl.pallas_call(
        _pre_body,
        grid=(_NBLK,),
        in_specs=[
            pl.BlockSpec((_BM, _IN), lambda i: (i, 0)),
            pl.BlockSpec((_IN, _HD0), lambda i: (0, 0)),
            pl.BlockSpec((_HD0, _H), lambda i: (0, 0)),
            pl.BlockSpec((_HD0, _H), lambda i: (0, 0)),
            pl.BlockSpec((_IN, _HD0), lambda i: (0, 0)),
        ],
        out_specs=[
            pl.BlockSpec((_BM, _HD0), lambda i: (i, 0)),
            pl.BlockSpec((_BM, _H), lambda i: (i, 0)),
            pl.BlockSpec((_BM, _H), lambda i: (i, 0)),
            pl.BlockSpec((_BM, _HD0), lambda i: (i, 0)),
        ],
        out_shape=[
            jax.ShapeDtypeStruct((_NP, _HD0), F32),
            jax.ShapeDtypeStruct((_NP, _H), F32),
            jax.ShapeDtypeStruct((_NP, _H), F32),
            jax.ShapeDtypeStruct((_NP, _HD0), F32),
        ],
    )(featp, W, Al, Ar, R)


def _mid_body(*refs):
    nums = refs[:8]
    den, res0, b0, W1, R1, Al1, Ar1, ft1_o, el1_o, er1_o, res1_o = refs[8:]
    ft1 = jnp.zeros((_BM, _HD1), F32)
    res1 = jnp.zeros((_BM, _HD1), F32)
    for q in range(8):
        h = q // 4
        d = den[:, h:h + 1]
        z = jnp.where(d > 0, nums[q][0] / jnp.where(d > 0, d, 1.0), 0.0)
        z = z + res0[:, 16 * q:16 * (q + 1)] + b0[0, 16 * q:16 * (q + 1)][None, :]
        z = jnp.maximum(z, 0.0)
        ft1 = ft1 + jnp.dot(z, W1[16 * q:16 * (q + 1), :],
                            preferred_element_type=F32, precision=PREC)
        res1 = res1 + jnp.dot(z, R1[16 * q:16 * (q + 1), :],
                              preferred_element_type=F32, precision=PREC)
    ft1_o[...] = ft1
    res1_o[...] = res1
    el1_o[...] = jnp.dot(ft1, Al1[...], preferred_element_type=F32, precision=PREC)
    er1_o[...] = jnp.dot(ft1, Ar1[...], preferred_element_type=F32, precision=PREC)


def _tc_mid(num0, den0, res0, b0, W1, R1, Al1, Ar1):
    specs = []
    for q in range(8):
        specs.append(pl.BlockSpec((1, _BM, 16), lambda i, q=q: (q, i, 0)))
    return pl.pallas_call(
        _mid_body,
        grid=(_NBLK,),
        in_specs=specs + [
            pl.BlockSpec((_BM, _H), lambda i: (i, 0)),
            pl.BlockSpec((_BM, _HD0), lambda i: (i, 0)),
            pl.BlockSpec((1, _HD0), lambda i: (0, 0)),
            pl.BlockSpec((_HD0, _HD1), lambda i: (0, 0)),
            pl.BlockSpec((_HD0, _HD1), lambda i: (0, 0)),
            pl.BlockSpec((_HD1, _H), lambda i: (0, 0)),
            pl.BlockSpec((_HD1, _H), lambda i: (0, 0)),
        ],
        out_specs=[
            pl.BlockSpec((_BM, _HD1), lambda i: (i, 0)),
            pl.BlockSpec((_BM, _H), lambda i: (i, 0)),
            pl.BlockSpec((_BM, _H), lambda i: (i, 0)),
            pl.BlockSpec((_BM, _HD1), lambda i: (i, 0)),
        ],
        out_shape=[
            jax.ShapeDtypeStruct((_NP, _HD1), F32),
            jax.ShapeDtypeStruct((_NP, _H), F32),
            jax.ShapeDtypeStruct((_NP, _H), F32),
            jax.ShapeDtypeStruct((_NP, _HD1), F32),
        ],
    )(*([num0] * 8), den0, res0, b0, W1, R1, Al1, Ar1)


def _post_body(n0, n1, den, res1, b1, out_o):
    nums = (n0, n1)
    parts = []
    for h in range(2):
        d = den[:, h:h + 1]
        v = jnp.where(d > 0, nums[h][0] / jnp.where(d > 0, d, 1.0), 0.0)
        v = v + res1[:, 16 * h:16 * (h + 1)] + b1[0, 16 * h:16 * (h + 1)][None, :]
        parts.append(v)
    out_o[...] = jnp.concatenate(parts, axis=1)


def _tc_post(num1, den1, res1, b1):
    specs = []
    for h in range(2):
        specs.append(pl.BlockSpec((1, _BM, 16), lambda i, h=h: (h, i, 0)))
    return pl.pallas_call(
        _post_body,
        grid=(_NBLK,),
        in_specs=specs + [
            pl.BlockSpec((_BM, _H), lambda i: (i, 0)),
            pl.BlockSpec((_BM, _HD1), lambda i: (i, 0)),
            pl.BlockSpec((1, _HD1), lambda i: (0, 0)),
        ],
        out_specs=pl.BlockSpec((_BM, _HD1), lambda i: (i, 0)),
        out_shape=jax.ShapeDtypeStruct((_NP, _HD1), F32),
    )(num1, num1, den1, res1, b1)


def _sc_layer(chunks_per_core, mult, halves):
    mesh = plsc.VectorSubcoreMesh(core_axis_name="c", subcore_axis_name="s",
                                  num_cores=_NC, num_subcores=_NS)
    nchunks = chunks_per_core * _NC
    rw = 16
    nph = _NP // halves
    rows_h = nph // _NS
    gpad = 8 if halves > 1 else 0
    zrows = 28

    def body(*refs):
        fts = refs[0:3]
        els = refs[3:6]
        ers = refs[6:9]
        srcs = refs[9:12]
        dsts = refs[12:15]
        num_os = refs[15:18]
        den_os = refs[18:21]
        (el_t, er_t, src_t, dst_t, dsc_t, w_t, rows_t, zb, zbd,
         num_sp, den_sp) = refs[21:]
        c = lax.axis_index("c")
        s = lax.axis_index("s")
        base_r = s * rows_h

        z16 = jnp.zeros((16,), F32)
        for r in range(zrows):
            zb[r, :] = z16
        for k in range(224 // 16):
            zbd[pl.ds(16 * k, 16)] = z16

        for v in range(_G):
            pltpu.sync_copy(els[v].at[c], el_t)
            pltpu.sync_copy(ers[v].at[c], er_t)

            def pass_body(t, carry):
                p = t // chunks_per_core
                j = t - p * chunks_per_core
                chunk = c * chunks_per_core + j

                def zrow(i, icarry):
                    pltpu.sync_copy(
                        zb, num_sp.at[pl.ds(base_r + i * zrows, zrows)])
                    return icarry
                lax.fori_loop(0, rows_h // zrows, zrow, 0)

                @pl.when(j == 0)
                def _():
                    for k in range(rows_h // 224):
                        pltpu.sync_copy(
                            zbd, den_sp.at[pl.ds(base_r + k * 224, 224)])
                plsc.subcore_barrier()

                def ebody(ib, icarry):
                    par = lax.rem(ib, 2)
                    poff = par * _B
                    qoff = _B - poff
                    ebase = s * _EDG_T + ib * _B
                    pltpu.sync_copy(srcs[v].at[pl.ds(ebase, _B)], src_t)
                    pltpu.sync_copy(dsts[v].at[pl.ds(ebase, _B)], dst_t)
                    for k in range(_B // 16):
                        sv = src_t[pl.ds(k * 16, 16)]
                        dv = dst_t[pl.ds(k * 16, 16)]
                        e = (plsc.load_gather(el_t, [sv])
                             + plsc.load_gather(er_t, [dv]))
                        e = jnp.where(e >= 0, e, 0.2 * e)
                        w_t[pl.ds(16 + k * 16, 16)] = jnp.exp(e)
                        if halves > 1:
                            rel = dv - p * nph
                            ok = (rel >= 0) & (rel < nph)
                            dsc_t[pl.ds(poff + k * 16, 16)] = jnp.where(
                                ok, rel, nph)
                        else:
                            dsc_t[pl.ds(poff + k * 16, 16)] = dv
                        adjv = sv * mult + chunk
                        pltpu.sync_copy(fts[v].at[adjv],
                                        rows_t.at[pl.ds(poff + k * 16, 16)])
                    for i2 in range(_B):
                        wv = plsc.load_gather(
                            w_t, [jnp.full((16,), 16 + i2, jnp.int32)])
                        rows_t[poff + i2, :] = rows_t[poff + i2, :] * wv

                    @pl.when(ib > 0)
                    def _():
                        pltpu.sync_copy(
                            rows_t.at[pl.ds(qoff, _B)],
                            num_sp.at[dsc_t.at[pl.ds(qoff, _B)]], add=True)

                    @pl.when(j == 0)
                    def _():
                        pltpu.sync_copy(
                            w_t.at[pl.ds(16, _B)], den_sp.at[dsc_t.at[pl.ds(poff, _B)]],
                            add=True)
                    return icarry
                lax.fori_loop(0, _NBATCH, ebody, 0)
                plsc.subcore_barrier()
                toff = ((_NBATCH - 1) % 2) * _B
                pltpu.sync_copy(rows_t.at[pl.ds(toff, _B)],
                                num_sp.at[dsc_t.at[pl.ds(toff, _B)]],
                                add=True)
                plsc.subcore_barrier()

                pltpu.sync_copy(
                    num_sp.at[pl.ds(base_r, rows_h)],
                    num_os[v].at[chunk, pl.ds(p * nph + base_r, rows_h)])

                @pl.when(j == 0)
                def _():
                    pltpu.sync_copy(
                        den_sp.at[pl.ds(base_r, rows_h)],
                        den_os[v].at[
                            pl.ds(c * _NP + p * nph + base_r, rows_h)])
                plsc.subcore_barrier()
                return carry
            lax.fori_loop(0, halves * chunks_per_core, pass_body, 0)

    return pl.kernel(
        body,
        compiler_params=pltpu.CompilerParams(needs_layout_passes=False,
                                             use_tc_tiling_on_sc=False),
        out_type=[jax.ShapeDtypeStruct((nchunks, _NP, rw), F32)] * _G
        + [jax.ShapeDtypeStruct((_NC * _NP,), F32)] * _G,
        mesh=mesh,
        scratch_types=[
            pltpu.VMEM((_NP,), F32),
            pltpu.VMEM((_NP,), F32),
            pltpu.VMEM((_B,), jnp.int32),
            pltpu.VMEM((_B,), jnp.int32),
            pltpu.VMEM((2 * _B,), jnp.int32),
            pltpu.VMEM((16 + _B,), F32),
            pltpu.VMEM((2 * _B, rw), F32),
            pltpu.VMEM((zrows, rw), F32),
            pltpu.VMEM((224,), F32),
            pltpu.VMEM_SHARED((nph + gpad, rw), F32),
            pltpu.VMEM_SHARED((nph + gpad,), F32),
        ],
    )


@functools.lru_cache(maxsize=None)
def _sc_layer_cached(chunks_per_core, mult, halves):
    return _sc_layer(chunks_per_core, mult, halves)


def _attn_mat(a, d):
    out = jnp.zeros((_H * d, _H), F32)
    for h in range(_H):
        out = out.at[h * d:(h + 1) * d, h].set(a[h])
    return out


def kernel(feat, edge_index_0, edge_index_1, edge_index_2,
           W0, al0, ar0, b0, R0, W1, al1, ar1, b1, R1):
    featp = jnp.pad(feat, ((0, _NP - _N), (0, 0)))
    edges = (edge_index_0, edge_index_1, edge_index_2)
    srcs = [e[0] for e in edges]
    dsts = [e[1] for e in edges]

    ft0s, el0s, er0s, res0s = [], [], [], []
    for i in range(_G):
        ft0, el0, er0, res0 = _tc_pre(featp, W0[i], _attn_mat(al0[i], _D0),
                                      _attn_mat(ar0[i], _D0), R0[i])
        ft0s.append(ft0.reshape(8 * _NP, 16))
        el0s.append(el0.T)
        er0s.append(er0.T)
        res0s.append(res0)

    r0 = _sc_layer_cached(4, 8, 2)(*ft0s, *el0s, *er0s, *srcs, *dsts)
    num0s, den0s = r0[:_G], r0[_G:]

    ft1s, el1s, er1s, res1s = [], [], [], []
    for i in range(_G):
        ft1, el1, er1, res1 = _tc_mid(
            num0s[i], den0s[i].reshape(_NC, _NP).T, res0s[i],
            b0[i].reshape(1, _HD0), W1[i], R1[i],
            _attn_mat(al1[i], _D1), _attn_mat(ar1[i], _D1))
        ft1s.append(ft1.reshape(2 * _NP, 16))
        el1s.append(el1.T)
        er1s.append(er1.T)
        res1s.append(res1)

    r1 = _sc_layer_cached(1, 2, 2)(*ft1s, *el1s, *er1s, *srcs, *dsts)
    num1s, den1s = r1[:_G], r1[_G:]

    outs = []
    for i in range(_G):
        out = _tc_post(num1s[i], den1s[i].reshape(_NC, _NP).T, res1s[i],
                       b1[i].reshape(1, _HD1))
        outs.append(out[:_N])
    return tuple(outs)
